# baseline (speedup 1.0000x reference)
import functools

import jax
import jax.numpy as jnp
from jax import lax
from jax.experimental import pallas as pl
from jax.experimental.pallas import tpu as pltpu
from jax.experimental.pallas import tpu_sc as plsc

N_FACTORS = 64
BATCH = 16384
NC, NS, L = 2, 16, 16
NW = NC * NS
B_PER_W = BATCH // NW
CHUNK = 128
N_CHUNKS = B_PER_W // CHUNK
GPC = CHUNK // L


def _sc_body(um_hbm, data_hbm, out_hbm,
             data_v, uidx_v, midx_v, u_bufs, m_bufs, out_v, sem):
    wid = lax.axis_index("s") * NC + lax.axis_index("c")
    base = wid * B_PER_W

    pltpu.sync_copy(data_hbm.at[wid], data_v)

    lane = lax.iota(jnp.int32, L)
    two_lane = lane * 2

    @plsc.parallel_loop(0, B_PER_W, L)
    def _deint(i):
        b2 = 2 * i + two_lane
        uidx_v[pl.ds(i, L)] = plsc.load_gather(data_v, [b2])
        midx_v[pl.ds(i, L)] = plsc.load_gather(data_v, [b2 + 1])

    def start_gather(c, buf):
        pltpu.make_async_copy(
            um_hbm.at[uidx_v.at[pl.ds(c * CHUNK, CHUNK)]], u_bufs.at[buf], sem
        ).start()
        pltpu.make_async_copy(
            um_hbm.at[midx_v.at[pl.ds(c * CHUNK, CHUNK)]], m_bufs.at[buf], sem
        ).start()

    def wait_gather(c, buf):
        pltpu.make_async_copy(
            um_hbm.at[uidx_v.at[pl.ds(c * CHUNK, CHUNK)]], u_bufs.at[buf], sem
        ).wait()
        pltpu.make_async_copy(
            um_hbm.at[midx_v.at[pl.ds(c * CHUNK, CHUNK)]], m_bufs.at[buf], sem
        ).wait()

    start_gather(0, 0)

    for c in range(N_CHUNKS):
        buf = c & 1
        if c + 1 < N_CHUNKS:
            start_gather(c + 1, (c + 1) & 1)
        wait_gather(c, buf)

        @plsc.parallel_loop(0, GPC, 1)
        def _group(g):
            gbase = c * CHUNK + g * L
            acc = jnp.zeros((L,), jnp.float32)
            for j in range(L):
                r = g * L + j
                parts = []
                for k in range(N_FACTORS // L):
                    uu = u_bufs[buf, r, pl.ds(k * L, L)]
                    mm = m_bufs[buf, r, pl.ds(N_FACTORS + k * L, L)]
                    parts.append(uu * mm)
                p = (parts[0] + parts[1]) + (parts[2] + parts[3])
                acc = jnp.where(lane == j, jnp.sum(p), acc)
            out_v[pl.ds(gbase, L)] = acc

    pltpu.sync_copy(out_v, out_hbm.at[pl.ds(base, B_PER_W)])


@jax.jit
def kernel(data, user_factors, movie_factors):
    um = jnp.concatenate([user_factors.T, movie_factors.T], axis=0).T
    data_r = data.reshape(NW, B_PER_W * 2)
    mesh = plsc.VectorSubcoreMesh(core_axis_name="c", subcore_axis_name="s")
    f = pl.kernel(
        _sc_body,
        out_type=jax.ShapeDtypeStruct((BATCH,), jnp.float32),
        mesh=mesh,
        scratch_types=[
            pltpu.VMEM((B_PER_W * 2,), jnp.int32),
            pltpu.VMEM((B_PER_W,), jnp.int32),
            pltpu.VMEM((B_PER_W,), jnp.int32),
            pltpu.VMEM((2, CHUNK, 2 * N_FACTORS), jnp.float32),
            pltpu.VMEM((2, CHUNK, 2 * N_FACTORS), jnp.float32),
            pltpu.VMEM((B_PER_W,), jnp.float32),
            pltpu.SemaphoreType.DMA,
        ],
        compiler_params=pltpu.CompilerParams(
            needs_layout_passes=False, use_tc_tiling_on_sc=True),
    )
    return f(um, data_r)


# speedup vs baseline: 1.0035x; 1.0035x over previous
import jax
import jax.numpy as jnp
from jax import lax
from jax.experimental import pallas as pl
from jax.experimental.pallas import tpu as pltpu
from jax.experimental.pallas import tpu_sc as plsc

N_FACTORS = 64
BATCH = 16384
NC, NS, L = 2, 16, 16
NW = NC * NS
B_PER_W = BATCH // NW
CHUNK = 128
N_CHUNKS = B_PER_W // CHUNK
GPC = CHUNK // L


def _sc_body(um_hbm, data_hbm, out_hbm,
             data_v, uidx_v, midx_v, u_bufs, m_bufs, out_v, sem):
    wid = lax.axis_index("s") * NC + lax.axis_index("c")
    base = wid * B_PER_W

    pltpu.sync_copy(data_hbm.at[wid], data_v)

    lane = lax.iota(jnp.int32, L)
    two_lane = lane * 2

    @plsc.parallel_loop(0, B_PER_W, L)
    def _deint(i):
        b2 = 2 * i + two_lane
        uidx_v[pl.ds(i, L)] = plsc.load_gather(data_v, [b2])
        midx_v[pl.ds(i, L)] = plsc.load_gather(data_v, [b2 + 1])

    def start_gather(c, buf):
        pltpu.make_async_copy(
            um_hbm.at[uidx_v.at[pl.ds(c * CHUNK, CHUNK)]], u_bufs.at[buf], sem
        ).start()
        pltpu.make_async_copy(
            um_hbm.at[midx_v.at[pl.ds(c * CHUNK, CHUNK)]], m_bufs.at[buf], sem
        ).start()

    def wait_gather(c, buf):
        pltpu.make_async_copy(
            um_hbm.at[uidx_v.at[pl.ds(c * CHUNK, CHUNK)]], u_bufs.at[buf], sem
        ).wait()
        pltpu.make_async_copy(
            um_hbm.at[midx_v.at[pl.ds(c * CHUNK, CHUNK)]], m_bufs.at[buf], sem
        ).wait()

    start_gather(0, 0)

    for c in range(N_CHUNKS):
        buf = c & 1
        if c + 1 < N_CHUNKS:
            start_gather(c + 1, (c + 1) & 1)
        wait_gather(c, buf)

        @plsc.parallel_loop(0, GPC, 1)
        def _group(g):
            gbase = c * CHUNK + g * L
            acc = jnp.zeros((L,), jnp.float32)
            for j in range(L):
                r = g * L + j
                parts = []
                for k in range(N_FACTORS // L):
                    uu = u_bufs[buf, r, pl.ds(k * L, L)]
                    mm = m_bufs[buf, r, pl.ds(N_FACTORS + k * L, L)]
                    parts.append(uu * mm)
                p = (parts[0] + parts[1]) + (parts[2] + parts[3])
                acc = jnp.where(lane == j, jnp.sum(p), acc)
            out_v[pl.ds(gbase, L)] = acc

    pltpu.sync_copy(out_v, out_hbm.at[pl.ds(base, B_PER_W)])


@jax.jit
def kernel(data, user_factors, movie_factors):
    um = jnp.concatenate([user_factors, movie_factors], axis=1)
    data_r = data.reshape(NW, B_PER_W * 2)
    mesh = plsc.VectorSubcoreMesh(core_axis_name="c", subcore_axis_name="s")
    f = pl.kernel(
        _sc_body,
        out_type=jax.ShapeDtypeStruct((BATCH,), jnp.float32),
        mesh=mesh,
        scratch_types=[
            pltpu.VMEM((B_PER_W * 2,), jnp.int32),
            pltpu.VMEM((B_PER_W,), jnp.int32),
            pltpu.VMEM((B_PER_W,), jnp.int32),
            pltpu.VMEM((2, CHUNK, 2 * N_FACTORS), jnp.float32),
            pltpu.VMEM((2, CHUNK, 2 * N_FACTORS), jnp.float32),
            pltpu.VMEM((B_PER_W,), jnp.float32),
            pltpu.SemaphoreType.DMA,
        ],
        compiler_params=pltpu.CompilerParams(
            needs_layout_passes=False, use_tc_tiling_on_sc=True),
    )
    return f(um, data_r)
